# baseline (speedup 1.0000x reference)
import numpy as np
import jax
import jax.numpy as jnp
from jax.experimental import pallas as pl
from jax.experimental.pallas import tpu as pltpu

LB = 1024
K1 = 224
K2 = 720


def _build_idx1():
    idx = np.full((2, 2, 2, 10, 12, 8, 28), 250, np.int32)
    for qi in range(2):
        for qj in range(2):
            for il in range(2):
                for jp in range(12):
                    for kh in range(5):
                        for kw in range(5):
                            rl = 2 * il + qi + kh
                            w = 2 * jp + qj + kw
                            idx[qi, qj, il, :, jp, rl, w] = (
                                np.arange(10) * 25 + kh * 5 + kw)
    return idx.reshape(960, 224)


def _build_idx2():
    idx = np.full((2, 2, 10, 4, 6, 10, 12), 2500, np.int32)
    for qi2 in range(2):
        for qj2 in range(2):
            for j2 in range(4):
                for kh in range(5):
                    for kw in range(5):
                        ilp = qi2 + kh
                        jp = 2 * j2 + qj2 + kw
                        for c in range(10):
                            idx[qi2, qj2, :, j2, ilp, c, jp] = (
                                np.arange(10) * 250 + c * 25 + kh * 5 + kw)
    return idx.reshape(160, 720)


_I1 = _build_idx1()
_I2 = _build_idx2()


def _fused_kernel(xt_ref, a1_ref, a2_ref, wf_ref, b1_ref, b2_ref, bf_ref,
                  logit_ref, feat_ref, y1_ref, acc_ref, y2_ref):
    for s in range(6):
        acc_ref[...] = jnp.dot(a1_ref[...], xt_ref[112 * s:112 * s + K1, :],
                               preferred_element_type=jnp.float32)
        m = jnp.maximum(jnp.maximum(acc_ref[0:240, :], acc_ref[240:480, :]),
                        jnp.maximum(acc_ref[480:720, :], acc_ref[720:960, :]))
        y1_ref[240 * s:240 * s + 240, :] = jnp.maximum(
            m + b1_ref[...], 0.0).astype(jnp.bfloat16)

    for t in range(4):
        acc_ref[0:160, :] = jnp.dot(a2_ref[...],
                                    y1_ref[240 * t:240 * t + K2, :],
                                    preferred_element_type=jnp.float32)
        m = jnp.maximum(jnp.maximum(acc_ref[0:40, :], acc_ref[40:80, :]),
                        jnp.maximum(acc_ref[80:120, :], acc_ref[120:160, :]))
        y2_ref[40 * t:40 * t + 40, :] = jnp.maximum(m + b2_ref[...], 0.0)

    logits = jnp.dot(wf_ref[...], y2_ref[...],
                     preferred_element_type=jnp.float32) + bf_ref[...]
    logit_ref[...] = logits.T

    y2p = jnp.transpose(y2_ref[...].reshape(4, 10, 4, LB),
                        (1, 0, 2, 3)).reshape(160, LB)
    feat_ref[...] = y2p.T


def _pack_a1(w1):
    w1z = jnp.concatenate([w1.reshape(250).astype(jnp.float32),
                           jnp.zeros((1,), jnp.float32)])
    return w1z[_I1]


def _pack_a2(w2):
    w2z = jnp.concatenate([w2.reshape(2500).astype(jnp.float32),
                           jnp.zeros((1,), jnp.float32)])
    return w2z[_I2]


def kernel(x, w1, b1, w2, b2, wf, bf):
    B = x.shape[0]
    Bp = ((B + LB - 1) // LB) * LB
    T = Bp // LB

    xt = x.reshape(B, 784).T.astype(jnp.bfloat16)
    if Bp != B:
        xt = jnp.pad(xt, ((0, 0), (0, Bp - B)))

    a1 = _pack_a1(w1).astype(jnp.bfloat16)
    a2 = _pack_a2(w2).astype(jnp.bfloat16)
    wfp = wf.astype(jnp.float32).reshape(10, 4, 4, 10).transpose(1, 0, 2, 3)
    wft = jnp.pad(wfp.reshape(160, 10).T, ((0, 6), (0, 0)))

    b1r = jnp.broadcast_to(b1.astype(jnp.float32)[None, :, None],
                           (2, 10, 12)).reshape(240, 1)
    b2r = jnp.broadcast_to(b2.astype(jnp.float32)[:, None],
                           (10, 4)).reshape(40, 1)
    bfr = jnp.pad(bf.astype(jnp.float32), (0, 6)).reshape(16, 1)

    logits_o, feat_o = pl.pallas_call(
        _fused_kernel,
        out_shape=(jax.ShapeDtypeStruct((Bp, 16), jnp.float32),
                   jax.ShapeDtypeStruct((Bp, 160), jnp.float32)),
        grid=(T,),
        in_specs=[pl.BlockSpec((784, LB), lambda t: (0, t)),
                  pl.BlockSpec((960, K1), lambda t: (0, 0)),
                  pl.BlockSpec((160, K2), lambda t: (0, 0)),
                  pl.BlockSpec((16, 160), lambda t: (0, 0)),
                  pl.BlockSpec((240, 1), lambda t: (0, 0)),
                  pl.BlockSpec((40, 1), lambda t: (0, 0)),
                  pl.BlockSpec((16, 1), lambda t: (0, 0))],
        out_specs=(pl.BlockSpec((LB, 16), lambda t: (t, 0)),
                   pl.BlockSpec((LB, 160), lambda t: (t, 0))),
        scratch_shapes=[pltpu.VMEM((1440, LB), jnp.bfloat16),
                        pltpu.VMEM((960, LB), jnp.float32),
                        pltpu.VMEM((160, LB), jnp.float32)],
        compiler_params=pltpu.CompilerParams(
            dimension_semantics=("parallel",),
            vmem_limit_bytes=64 * 1024 * 1024),
    )(xt, a1, a2, wft, b1r, b2r, bfr)

    return logits_o[:B, :10], feat_o[:B]


# speedup vs baseline: 15.6463x; 15.6463x over previous
import numpy as np
import jax
import jax.numpy as jnp
from jax.experimental import pallas as pl
from jax.experimental.pallas import tpu as pltpu

LB = 1024
K1 = 224
K2 = 720

_P1 = np.zeros((2, 2, 5, 8), np.float32)
for qi in range(2):
    for il in range(2):
        for kh in range(5):
            _P1[qi, il, kh, 2 * il + qi + kh] = 1.0
_Q1 = np.zeros((2, 12, 5, 28), np.float32)
for qj in range(2):
    for jp in range(12):
        for kw in range(5):
            _Q1[qj, jp, kw, 2 * jp + qj + kw] = 1.0
_P2 = np.zeros((2, 5, 6), np.float32)
for qi2 in range(2):
    for kh in range(5):
        _P2[qi2, kh, qi2 + kh] = 1.0
_Q2 = np.zeros((2, 4, 5, 12), np.float32)
for qj2 in range(2):
    for j2 in range(4):
        for kw in range(5):
            _Q2[qj2, j2, kw, 2 * j2 + qj2 + kw] = 1.0


def _fused_kernel(xt_ref, a1_ref, a2_ref, wf_ref, b1_ref, b2_ref, bf_ref,
                  logit_ref, feat_ref, y1_ref, acc_ref, y2_ref):
    for s in range(6):
        acc_ref[...] = jnp.dot(a1_ref[...], xt_ref[112 * s:112 * s + K1, :],
                               preferred_element_type=jnp.float32)
        m = jnp.maximum(jnp.maximum(acc_ref[0:240, :], acc_ref[240:480, :]),
                        jnp.maximum(acc_ref[480:720, :], acc_ref[720:960, :]))
        y1_ref[240 * s:240 * s + 240, :] = jnp.maximum(
            m + b1_ref[...], 0.0).astype(jnp.bfloat16)

    for t in range(4):
        acc_ref[0:160, :] = jnp.dot(a2_ref[...],
                                    y1_ref[240 * t:240 * t + K2, :],
                                    preferred_element_type=jnp.float32)
        m = jnp.maximum(jnp.maximum(acc_ref[0:40, :], acc_ref[40:80, :]),
                        jnp.maximum(acc_ref[80:120, :], acc_ref[120:160, :]))
        y2_ref[40 * t:40 * t + 40, :] = jnp.maximum(m + b2_ref[...], 0.0)

    logits = jnp.dot(wf_ref[...], y2_ref[...],
                     preferred_element_type=jnp.float32) + bf_ref[...]
    logit_ref[...] = logits.T

    y2p = jnp.transpose(y2_ref[...].reshape(4, 10, 4, LB),
                        (1, 0, 2, 3)).reshape(160, LB)
    feat_ref[...] = y2p.T


def _pack_a1(w1):
    w1m = w1.reshape(10, 5, 5).astype(jnp.float32)
    a1 = jnp.einsum('cuv,aiur,bjvw->abicjrw', w1m, _P1, _Q1)
    return a1.reshape(960, K1)


def _pack_a2(w2):
    w2m = w2.astype(jnp.float32)
    a2 = jnp.einsum('ocuv,aue,bjvp->abojecp', w2m, _P2, _Q2)
    return a2.reshape(160, K2)


def kernel(x, w1, b1, w2, b2, wf, bf):
    B = x.shape[0]
    Bp = ((B + LB - 1) // LB) * LB
    T = Bp // LB

    xt = x.reshape(B, 784).T.astype(jnp.bfloat16)
    if Bp != B:
        xt = jnp.pad(xt, ((0, 0), (0, Bp - B)))

    a1 = _pack_a1(w1).astype(jnp.bfloat16)
    a2 = _pack_a2(w2).astype(jnp.bfloat16)
    wfp = wf.astype(jnp.float32).reshape(10, 4, 4, 10).transpose(1, 0, 2, 3)
    wft = jnp.pad(wfp.reshape(160, 10).T, ((0, 6), (0, 0)))

    b1r = jnp.broadcast_to(b1.astype(jnp.float32)[None, :, None],
                           (2, 10, 12)).reshape(240, 1)
    b2r = jnp.broadcast_to(b2.astype(jnp.float32)[:, None],
                           (10, 4)).reshape(40, 1)
    bfr = jnp.pad(bf.astype(jnp.float32), (0, 6)).reshape(16, 1)

    logits_o, feat_o = pl.pallas_call(
        _fused_kernel,
        out_shape=(jax.ShapeDtypeStruct((Bp, 16), jnp.float32),
                   jax.ShapeDtypeStruct((Bp, 160), jnp.float32)),
        grid=(T,),
        in_specs=[pl.BlockSpec((784, LB), lambda t: (0, t)),
                  pl.BlockSpec((960, K1), lambda t: (0, 0)),
                  pl.BlockSpec((160, K2), lambda t: (0, 0)),
                  pl.BlockSpec((16, 160), lambda t: (0, 0)),
                  pl.BlockSpec((240, 1), lambda t: (0, 0)),
                  pl.BlockSpec((40, 1), lambda t: (0, 0)),
                  pl.BlockSpec((16, 1), lambda t: (0, 0))],
        out_specs=(pl.BlockSpec((LB, 16), lambda t: (t, 0)),
                   pl.BlockSpec((LB, 160), lambda t: (t, 0))),
        scratch_shapes=[pltpu.VMEM((1440, LB), jnp.bfloat16),
                        pltpu.VMEM((960, LB), jnp.float32),
                        pltpu.VMEM((160, LB), jnp.float32)],
        compiler_params=pltpu.CompilerParams(
            dimension_semantics=("parallel",),
            vmem_limit_bytes=64 * 1024 * 1024),
    )(xt, a1, a2, wft, b1r, b2r, bfr)

    return logits_o[:B, :10], feat_o[:B]


# speedup vs baseline: 15.7274x; 1.0052x over previous
import numpy as np
import jax
import jax.numpy as jnp
from jax.experimental import pallas as pl
from jax.experimental.pallas import tpu as pltpu

LB = 2048
K1 = 224
K2 = 720

_P1 = np.zeros((2, 2, 5, 8), np.float32)
for qi in range(2):
    for il in range(2):
        for kh in range(5):
            _P1[qi, il, kh, 2 * il + qi + kh] = 1.0
_Q1 = np.zeros((2, 12, 5, 28), np.float32)
for qj in range(2):
    for jp in range(12):
        for kw in range(5):
            _Q1[qj, jp, kw, 2 * jp + qj + kw] = 1.0
_P2 = np.zeros((2, 5, 6), np.float32)
for qi2 in range(2):
    for kh in range(5):
        _P2[qi2, kh, qi2 + kh] = 1.0
_Q2 = np.zeros((2, 4, 5, 12), np.float32)
for qj2 in range(2):
    for j2 in range(4):
        for kw in range(5):
            _Q2[qj2, j2, kw, 2 * j2 + qj2 + kw] = 1.0


def _fused_kernel(xt_ref, a1_ref, a2_ref, wf_ref, b1_ref, b2_ref, bf_ref,
                  logit_ref, feat_ref, y1_ref, acc_ref, y2_ref):
    for s in range(6):
        acc_ref[...] = jnp.dot(a1_ref[...], xt_ref[112 * s:112 * s + K1, :],
                               preferred_element_type=jnp.float32)
        m = jnp.maximum(jnp.maximum(acc_ref[0:240, :], acc_ref[240:480, :]),
                        jnp.maximum(acc_ref[480:720, :], acc_ref[720:960, :]))
        y1_ref[240 * s:240 * s + 240, :] = jnp.maximum(
            m + b1_ref[...], 0.0).astype(jnp.bfloat16)

    for t in range(4):
        acc_ref[0:160, :] = jnp.dot(a2_ref[...],
                                    y1_ref[240 * t:240 * t + K2, :],
                                    preferred_element_type=jnp.float32)
        m = jnp.maximum(jnp.maximum(acc_ref[0:40, :], acc_ref[40:80, :]),
                        jnp.maximum(acc_ref[80:120, :], acc_ref[120:160, :]))
        y2_ref[40 * t:40 * t + 40, :] = jnp.maximum(m + b2_ref[...], 0.0)

    logits = jnp.dot(wf_ref[...], y2_ref[...],
                     preferred_element_type=jnp.float32) + bf_ref[...]
    logit_ref[...] = logits.T

    y2p = jnp.transpose(y2_ref[...].reshape(4, 10, 4, LB),
                        (1, 0, 2, 3)).reshape(160, LB)
    feat_ref[...] = y2p.T


def _pack_a1(w1):
    w1m = w1.reshape(10, 5, 5).astype(jnp.float32)
    a1 = jnp.einsum('cuv,aiur,bjvw->abicjrw', w1m, _P1, _Q1)
    return a1.reshape(960, K1)


def _pack_a2(w2):
    w2m = w2.astype(jnp.float32)
    a2 = jnp.einsum('ocuv,aue,bjvp->abojecp', w2m, _P2, _Q2)
    return a2.reshape(160, K2)


def kernel(x, w1, b1, w2, b2, wf, bf):
    B = x.shape[0]
    Bp = ((B + LB - 1) // LB) * LB
    T = Bp // LB

    xt = x.reshape(B, 784).T.astype(jnp.bfloat16)
    if Bp != B:
        xt = jnp.pad(xt, ((0, 0), (0, Bp - B)))

    a1 = _pack_a1(w1).astype(jnp.bfloat16)
    a2 = _pack_a2(w2).astype(jnp.bfloat16)
    wfp = wf.astype(jnp.float32).reshape(10, 4, 4, 10).transpose(1, 0, 2, 3)
    wft = jnp.pad(wfp.reshape(160, 10).T, ((0, 6), (0, 0)))

    b1r = jnp.broadcast_to(b1.astype(jnp.float32)[None, :, None],
                           (2, 10, 12)).reshape(240, 1)
    b2r = jnp.broadcast_to(b2.astype(jnp.float32)[:, None],
                           (10, 4)).reshape(40, 1)
    bfr = jnp.pad(bf.astype(jnp.float32), (0, 6)).reshape(16, 1)

    logits_o, feat_o = pl.pallas_call(
        _fused_kernel,
        out_shape=(jax.ShapeDtypeStruct((Bp, 16), jnp.float32),
                   jax.ShapeDtypeStruct((Bp, 160), jnp.float32)),
        grid=(T,),
        in_specs=[pl.BlockSpec((784, LB), lambda t: (0, t)),
                  pl.BlockSpec((960, K1), lambda t: (0, 0)),
                  pl.BlockSpec((160, K2), lambda t: (0, 0)),
                  pl.BlockSpec((16, 160), lambda t: (0, 0)),
                  pl.BlockSpec((240, 1), lambda t: (0, 0)),
                  pl.BlockSpec((40, 1), lambda t: (0, 0)),
                  pl.BlockSpec((16, 1), lambda t: (0, 0))],
        out_specs=(pl.BlockSpec((LB, 16), lambda t: (t, 0)),
                   pl.BlockSpec((LB, 160), lambda t: (t, 0))),
        scratch_shapes=[pltpu.VMEM((1440, LB), jnp.bfloat16),
                        pltpu.VMEM((960, LB), jnp.float32),
                        pltpu.VMEM((160, LB), jnp.float32)],
        compiler_params=pltpu.CompilerParams(
            dimension_semantics=("parallel",),
            vmem_limit_bytes=64 * 1024 * 1024),
    )(xt, a1, a2, wft, b1r, b2r, bfr)

    return logits_o[:B, :10], feat_o[:B]
